# baseline (speedup 1.0000x reference)
import jax
import jax.numpy as jnp
from jax.experimental import pallas as pl

N_NODES = 10000
NGRAPH = 128
L = 3


def _bn(h, g, b):
    m = h.mean(0)
    v = h.var(0)
    return g * (h - m) / jnp.sqrt(v + 1e-5) + b


def _mlp(params, name, h, nch):
    for j in range(1, nch):
        h = h @ params[name + '_W%d' % j] + params[name + '_b%d' % j]
        if j < nch - 1:
            h = _bn(h, params[name + '_g%d' % j], params[name + '_be%d' % j])
            h = jax.nn.relu(h)
    return h


def _gin(params, ext, x, src, dst, n):
    h = x
    for l in range(L):
        agg = jax.ops.segment_sum(h[src], dst, num_segments=n)
        h2 = (1.0 + params[ext + '_eps_%d' % l]) * h + agg
        h2 = jax.nn.relu(h2 @ params[ext + '_W1_%d' % l] + params[ext + '_b1_%d' % l])
        h = jax.nn.relu(h2 @ params[ext + '_W2_%d' % l] + params[ext + '_b2_%d' % l])
    return h


def _mean_pool(h, batch, g):
    s = jax.ops.segment_sum(h, batch, num_segments=g)
    c = jax.ops.segment_sum(jnp.ones((h.shape[0],), h.dtype), batch, num_segments=g)
    return s / jnp.maximum(c, 1.0)[:, None]


def _softmax_pallas(logits):
    def body(x_ref, o_ref):
        x = x_ref[...]
        m = jnp.max(x, axis=-1, keepdims=True)
        e = jnp.exp(x - m)
        o_ref[...] = e / jnp.sum(e, axis=-1, keepdims=True)

    return pl.pallas_call(
        body, out_shape=jax.ShapeDtypeStruct(logits.shape, logits.dtype)
    )(logits)


def kernel(x, edge_index, batch, params):
    src = edge_index[0]
    dst = edge_index[1]
    hfg = _gin(params, 'fg', x, src, dst, N_NODES)
    gfg = _mean_pool(hfg, batch, NGRAPH)
    logits = _mlp(params, 'outmlp', gfg, 4)
    y_prob = _softmax_pallas(logits)
    y_pred = y_prob
    heg = _gin(params, 'eg', x, src, dst, N_NODES)
    geg = _mean_pool(heg, batch, NGRAPH)
    graph_repr = jax.nn.sigmoid(_mlp(params, 'lfenv', jnp.concatenate([geg, y_pred], axis=-1), 3))
    hig = _gin(params, 'ig', x, src, dst, N_NODES)
    node_in = jnp.concatenate([hig, y_pred[batch]], axis=-1)
    node_score = jax.nn.sigmoid(_mlp(params, 'lfinv', node_in, 3))
    env_pred = _mlp(params, 'envp', graph_repr, 3)
    return (y_pred, y_prob, graph_repr, node_score, env_pred)


# speedup vs baseline: 1.0069x; 1.0069x over previous
import functools

import jax
import jax.numpy as jnp
from jax.experimental import pallas as pl
from jax.experimental.pallas import tpu as pltpu

NGRAPH = 128
L = 3
BR = 1000


def _edge_segment_sum(h, src, dst, n):
    return jax.ops.segment_sum(h[src], dst, num_segments=n)


def _gin_layer_body(h_ref, agg_ref, w1_ref, b1_ref, w2_ref, b2_ref, eps_ref,
                    o_ref):
    z = (1.0 + eps_ref[0]) * h_ref[...] + agg_ref[...]
    h2 = jnp.maximum(_mm(z, w1_ref[...]) + b1_ref[...], 0.0)
    o_ref[...] = jnp.maximum(_mm(h2, w2_ref[...]) + b2_ref[...], 0.0)


def _gin_layer(h, agg, eps, w1, b1, w2, b2):
    n, d = h.shape
    hh = w1.shape[1]
    nb = n // BR
    return pl.pallas_call(
        _gin_layer_body,
        grid=(nb,),
        in_specs=[
            pl.BlockSpec((BR, d), lambda i: (i, 0)),
            pl.BlockSpec((BR, d), lambda i: (i, 0)),
            pl.BlockSpec((d, hh), lambda i: (0, 0)),
            pl.BlockSpec((1, hh), lambda i: (0, 0)),
            pl.BlockSpec((hh, hh), lambda i: (0, 0)),
            pl.BlockSpec((1, hh), lambda i: (0, 0)),
            pl.BlockSpec(memory_space=pltpu.SMEM),
        ],
        out_specs=pl.BlockSpec((BR, hh), lambda i: (i, 0)),
        out_shape=jax.ShapeDtypeStruct((n, hh), jnp.float32),
    )(h, agg, w1, b1.reshape(1, hh), w2, b2.reshape(1, hh),
      eps.reshape(1).astype(jnp.float32))


def _pool_body(h_ref, b3_ref, s_ref, c_ref):
    i = pl.program_id(0)
    bvec = b3_ref[0, 0, :]
    onehot = (bvec[:, None] == jax.lax.broadcasted_iota(
        jnp.int32, (1, NGRAPH), 1)).astype(jnp.float32)

    @pl.when(i == 0)
    def _():
        s_ref[...] = jnp.zeros_like(s_ref)
        c_ref[...] = jnp.zeros_like(c_ref)

    s_ref[...] += jax.lax.dot_general(
        onehot, h_ref[...], (((0,), (0,)), ((), ())),
        preferred_element_type=jnp.float32,
        precision=jax.lax.Precision.HIGHEST)
    c_ref[...] += jax.lax.dot_general(
        onehot, jnp.ones((onehot.shape[0], 1), jnp.float32),
        (((0,), (0,)), ((), ())), preferred_element_type=jnp.float32,
        precision=jax.lax.Precision.HIGHEST)


def _pool_sums(h, batch3):
    n, d = h.shape
    nb = n // BR
    return pl.pallas_call(
        _pool_body,
        grid=(nb,),
        in_specs=[
            pl.BlockSpec((BR, d), lambda i: (i, 0)),
            pl.BlockSpec((1, 1, BR), lambda i: (i, 0, 0)),
        ],
        out_specs=[
            pl.BlockSpec((NGRAPH, d), lambda i: (0, 0)),
            pl.BlockSpec((NGRAPH, 1), lambda i: (0, 0)),
        ],
        out_shape=[
            jax.ShapeDtypeStruct((NGRAPH, d), jnp.float32),
            jax.ShapeDtypeStruct((NGRAPH, 1), jnp.float32),
        ],
    )(h, batch3)


def _bn_relu(h, g, be):
    m = jnp.mean(h, axis=0, keepdims=True)
    v = jnp.mean((h - m) * (h - m), axis=0, keepdims=True)
    return jnp.maximum(g * (h - m) / jnp.sqrt(v + 1e-5) + be, 0.0)


def _mm(a, w):
    return jnp.dot(a, w, preferred_element_type=jnp.float32)


def _mm_exact(a, w):
    return jnp.dot(a, w, preferred_element_type=jnp.float32,
                   precision=jax.lax.Precision.HIGHEST)


def _outmlp_body(s_ref, c_ref, w1, b1, g1, be1, w2, b2, g2, be2, w3, b3,
                 y_ref):
    gfg = s_ref[...] / jnp.maximum(c_ref[...], 1.0)
    h = _bn_relu(_mm(gfg, w1[...]) + b1[...], g1[...], be1[...])
    h = _bn_relu(_mm(h, w2[...]) + b2[...], g2[...], be2[...])
    logits = _mm(h, w3[...]) + b3[...]
    m = jnp.max(logits, axis=-1, keepdims=True)
    e = jnp.exp(logits - m)
    y_ref[...] = e / jnp.sum(e, axis=-1, keepdims=True)


def _outmlp(s, c, p):
    args = (s, c,
            p['outmlp_W1'], p['outmlp_b1'].reshape(1, -1),
            p['outmlp_g1'].reshape(1, -1), p['outmlp_be1'].reshape(1, -1),
            p['outmlp_W2'], p['outmlp_b2'].reshape(1, -1),
            p['outmlp_g2'].reshape(1, -1), p['outmlp_be2'].reshape(1, -1),
            p['outmlp_W3'], p['outmlp_b3'].reshape(1, -1))
    return pl.pallas_call(
        _outmlp_body,
        out_shape=jax.ShapeDtypeStruct((NGRAPH, p['outmlp_W3'].shape[1]),
                                       jnp.float32),
    )(*args)


def _heads_body(s_ref, c_ref, y_ref, w1a, w1b, b1, g1, be1, w2, b2,
                ew1, eb1, eg1, ebe1, ew2, eb2, gr_ref, env_ref):
    geg = s_ref[...] / jnp.maximum(c_ref[...], 1.0)
    h = _mm(geg, w1a[...]) + _mm(y_ref[...], w1b[...]) + b1[...]
    h = _bn_relu(h, g1[...], be1[...])
    gr = jax.nn.sigmoid(_mm(h, w2[...]) + b2[...])
    gr_ref[...] = gr
    he = _bn_relu(_mm(gr, ew1[...]) + eb1[...], eg1[...], ebe1[...])
    env_ref[...] = _mm(he, ew2[...]) + eb2[...]


def _graph_heads(s, c, y, p):
    hdim = p['lfenv_W2'].shape[1]
    w1 = p['lfenv_W1']
    args = (s, c, y,
            w1[:hdim], w1[hdim:], p['lfenv_b1'].reshape(1, -1),
            p['lfenv_g1'].reshape(1, -1), p['lfenv_be1'].reshape(1, -1),
            p['lfenv_W2'], p['lfenv_b2'].reshape(1, -1),
            p['envp_W1'], p['envp_b1'].reshape(1, -1),
            p['envp_g1'].reshape(1, -1), p['envp_be1'].reshape(1, -1),
            p['envp_W2'], p['envp_b2'].reshape(1, -1))
    return pl.pallas_call(
        _heads_body,
        out_shape=[
            jax.ShapeDtypeStruct((NGRAPH, hdim), jnp.float32),
            jax.ShapeDtypeStruct((NGRAPH, p['envp_W2'].shape[1]),
                                 jnp.float32),
        ],
    )(*args)


def _lfinv_a_body(h_ref, b3_ref, y_ref, w1a, w1b, b1, h1_ref, ss_ref,
                  sq_ref):
    i = pl.program_id(0)
    bvec = b3_ref[0, 0, :]
    onehot = (bvec[:, None] == jax.lax.broadcasted_iota(
        jnp.int32, (1, NGRAPH), 1)).astype(jnp.float32)
    yb = _mm_exact(onehot, y_ref[...])
    h1 = _mm(h_ref[...], w1a[...]) + _mm(yb, w1b[...]) + b1[...]
    h1_ref[...] = h1

    @pl.when(i == 0)
    def _():
        ss_ref[...] = jnp.zeros_like(ss_ref)
        sq_ref[...] = jnp.zeros_like(sq_ref)

    ss_ref[...] += jnp.sum(h1, axis=0, keepdims=True)
    sq_ref[...] += jnp.sum(h1 * h1, axis=0, keepdims=True)


def _lfinv_a(h, batch3, y, p):
    n, d = h.shape
    w1 = p['lfinv_W1']
    hh = w1.shape[1]
    nb = n // BR
    return pl.pallas_call(
        _lfinv_a_body,
        grid=(nb,),
        in_specs=[
            pl.BlockSpec((BR, d), lambda i: (i, 0)),
            pl.BlockSpec((1, 1, BR), lambda i: (i, 0, 0)),
            pl.BlockSpec((NGRAPH, y.shape[1]), lambda i: (0, 0)),
            pl.BlockSpec((d, hh), lambda i: (0, 0)),
            pl.BlockSpec((y.shape[1], hh), lambda i: (0, 0)),
            pl.BlockSpec((1, hh), lambda i: (0, 0)),
        ],
        out_specs=[
            pl.BlockSpec((BR, hh), lambda i: (i, 0)),
            pl.BlockSpec((1, hh), lambda i: (0, 0)),
            pl.BlockSpec((1, hh), lambda i: (0, 0)),
        ],
        out_shape=[
            jax.ShapeDtypeStruct((n, hh), jnp.float32),
            jax.ShapeDtypeStruct((1, hh), jnp.float32),
            jax.ShapeDtypeStruct((1, hh), jnp.float32),
        ],
    )(h, batch3, y, w1[:d], w1[d:], p['lfinv_b1'].reshape(1, hh))


def _lfinv_b_body(h1_ref, ss_ref, sq_ref, g1, be1, w2, b2, o_ref, *,
                  n_rows):
    m = ss_ref[...] / n_rows
    v = sq_ref[...] / n_rows - m * m
    hn = jnp.maximum(g1[...] * (h1_ref[...] - m) / jnp.sqrt(v + 1e-5)
                     + be1[...], 0.0)
    o_ref[...] = jax.nn.sigmoid(_mm(hn, w2[...]) + b2[...])


def _lfinv_b(h1, ss, sq, p):
    n, hh = h1.shape
    w2 = p['lfinv_W2']
    dout = w2.shape[1]
    nb = n // BR
    return pl.pallas_call(
        functools.partial(_lfinv_b_body, n_rows=float(n)),
        grid=(nb,),
        in_specs=[
            pl.BlockSpec((BR, hh), lambda i: (i, 0)),
            pl.BlockSpec((1, hh), lambda i: (0, 0)),
            pl.BlockSpec((1, hh), lambda i: (0, 0)),
            pl.BlockSpec((1, hh), lambda i: (0, 0)),
            pl.BlockSpec((1, hh), lambda i: (0, 0)),
            pl.BlockSpec((hh, dout), lambda i: (0, 0)),
            pl.BlockSpec((1, dout), lambda i: (0, 0)),
        ],
        out_specs=pl.BlockSpec((BR, dout), lambda i: (i, 0)),
        out_shape=jax.ShapeDtypeStruct((n, dout), jnp.float32),
    )(h1, ss, sq, p['lfinv_g1'].reshape(1, hh), p['lfinv_be1'].reshape(1, hh),
      w2, p['lfinv_b2'].reshape(1, dout))


def _gin(params, ext, x, src, dst, n):
    h = x
    for l in range(L):
        agg = _edge_segment_sum(h, src, dst, n)
        h = _gin_layer(h, agg, params[ext + '_eps_%d' % l],
                       params[ext + '_W1_%d' % l], params[ext + '_b1_%d' % l],
                       params[ext + '_W2_%d' % l], params[ext + '_b2_%d' % l])
    return h


def kernel(x, edge_index, batch, params):
    n = x.shape[0]
    src = edge_index[0]
    dst = edge_index[1]
    batch3 = batch.astype(jnp.int32).reshape(n // BR, 1, BR)

    hfg = _gin(params, 'fg', x, src, dst, n)
    s_fg, c = _pool_sums(hfg, batch3)
    y = _outmlp(s_fg, c, params)

    heg = _gin(params, 'eg', x, src, dst, n)
    s_eg, _ = _pool_sums(heg, batch3)
    graph_repr, env_pred = _graph_heads(s_eg, c, y, params)

    hig = _gin(params, 'ig', x, src, dst, n)
    h1, ss, sq = _lfinv_a(hig, batch3, y, params)
    node_score = _lfinv_b(h1, ss, sq, params)

    return (y, y, graph_repr, node_score, env_pred)


# speedup vs baseline: 1.0252x; 1.0181x over previous
import functools

import jax
import jax.numpy as jnp
from jax.experimental import pallas as pl
from jax.experimental.pallas import tpu as pltpu

NGRAPH = 128
L = 3
BR = 1000


def _edge_segment_sum(h, src, dst, n):
    return jax.ops.segment_sum(h[src], dst, num_segments=n)


def _gin3_layer_body(h_ref, agg_ref, w1_ref, b1_ref, w2_ref, b2_ref,
                     eps_ref, o_ref):
    k = pl.program_id(1)
    z = (1.0 + eps_ref[k]) * h_ref[...] + agg_ref[...]
    h2 = jnp.maximum(_mm(z, w1_ref[0]) + b1_ref[0, 0], 0.0)
    o_ref[...] = jnp.maximum(_mm(h2, w2_ref[0]) + b2_ref[0, 0], 0.0)


def _gin3_layer(h, agg, eps3, w13, b13, w23, b23, shared_in):
    n = h.shape[0]
    d = w13.shape[1]
    hh = w13.shape[2]
    nb = n // BR
    if shared_in:
        hmap = lambda i, k: (i, 0)
    else:
        hmap = lambda i, k: (i, k)
    return pl.pallas_call(
        _gin3_layer_body,
        grid=(nb, 3),
        in_specs=[
            pl.BlockSpec((BR, d), hmap),
            pl.BlockSpec((BR, d), hmap),
            pl.BlockSpec((1, d, hh), lambda i, k: (k, 0, 0)),
            pl.BlockSpec((1, 1, hh), lambda i, k: (k, 0, 0)),
            pl.BlockSpec((1, hh, hh), lambda i, k: (k, 0, 0)),
            pl.BlockSpec((1, 1, hh), lambda i, k: (k, 0, 0)),
            pl.BlockSpec(memory_space=pltpu.SMEM),
        ],
        out_specs=pl.BlockSpec((BR, hh), lambda i, k: (i, k)),
        out_shape=jax.ShapeDtypeStruct((n, 3 * hh), jnp.float32),
    )(h, agg, w13, b13.reshape(3, 1, hh), w23, b23.reshape(3, 1, hh), eps3)


def _pool_body(h_ref, b3_ref, s_ref, c_ref):
    i = pl.program_id(0)
    bvec = b3_ref[0, 0, :]
    onehot = (bvec[:, None] == jax.lax.broadcasted_iota(
        jnp.int32, (1, NGRAPH), 1)).astype(jnp.float32)

    @pl.when(i == 0)
    def _():
        s_ref[...] = jnp.zeros_like(s_ref)
        c_ref[...] = jnp.zeros_like(c_ref)

    s_ref[...] += jax.lax.dot_general(
        onehot, h_ref[...], (((0,), (0,)), ((), ())),
        preferred_element_type=jnp.float32,
        precision=jax.lax.Precision.HIGHEST)
    c_ref[...] += jax.lax.dot_general(
        onehot, jnp.ones((onehot.shape[0], 1), jnp.float32),
        (((0,), (0,)), ((), ())), preferred_element_type=jnp.float32,
        precision=jax.lax.Precision.HIGHEST)


def _pool_sums(h, batch3):
    n, d = h.shape
    nb = n // BR
    return pl.pallas_call(
        _pool_body,
        grid=(nb,),
        in_specs=[
            pl.BlockSpec((BR, d), lambda i: (i, 0)),
            pl.BlockSpec((1, 1, BR), lambda i: (i, 0, 0)),
        ],
        out_specs=[
            pl.BlockSpec((NGRAPH, d), lambda i: (0, 0)),
            pl.BlockSpec((NGRAPH, 1), lambda i: (0, 0)),
        ],
        out_shape=[
            jax.ShapeDtypeStruct((NGRAPH, d), jnp.float32),
            jax.ShapeDtypeStruct((NGRAPH, 1), jnp.float32),
        ],
    )(h, batch3)


def _bn_relu(h, g, be):
    m = jnp.mean(h, axis=0, keepdims=True)
    v = jnp.mean((h - m) * (h - m), axis=0, keepdims=True)
    return jnp.maximum(g * (h - m) / jnp.sqrt(v + 1e-5) + be, 0.0)


def _mm(a, w):
    return jnp.dot(a, w, preferred_element_type=jnp.float32)


def _mm_exact(a, w):
    return jnp.dot(a, w, preferred_element_type=jnp.float32,
                   precision=jax.lax.Precision.HIGHEST)


def _outmlp_body(s_ref, c_ref, w1, b1, g1, be1, w2, b2, g2, be2, w3, b3,
                 y_ref):
    gfg = s_ref[...] / jnp.maximum(c_ref[...], 1.0)
    h = _bn_relu(_mm(gfg, w1[...]) + b1[...], g1[...], be1[...])
    h = _bn_relu(_mm(h, w2[...]) + b2[...], g2[...], be2[...])
    logits = _mm(h, w3[...]) + b3[...]
    m = jnp.max(logits, axis=-1, keepdims=True)
    e = jnp.exp(logits - m)
    y_ref[...] = e / jnp.sum(e, axis=-1, keepdims=True)


def _outmlp(s, c, p):
    args = (s, c,
            p['outmlp_W1'], p['outmlp_b1'].reshape(1, -1),
            p['outmlp_g1'].reshape(1, -1), p['outmlp_be1'].reshape(1, -1),
            p['outmlp_W2'], p['outmlp_b2'].reshape(1, -1),
            p['outmlp_g2'].reshape(1, -1), p['outmlp_be2'].reshape(1, -1),
            p['outmlp_W3'], p['outmlp_b3'].reshape(1, -1))
    return pl.pallas_call(
        _outmlp_body,
        out_shape=jax.ShapeDtypeStruct((NGRAPH, p['outmlp_W3'].shape[1]),
                                       jnp.float32),
    )(*args)


def _heads_body(s_ref, c_ref, y_ref, w1a, w1b, b1, g1, be1, w2, b2,
                ew1, eb1, eg1, ebe1, ew2, eb2, gr_ref, env_ref):
    geg = s_ref[...] / jnp.maximum(c_ref[...], 1.0)
    h = _mm(geg, w1a[...]) + _mm(y_ref[...], w1b[...]) + b1[...]
    h = _bn_relu(h, g1[...], be1[...])
    gr = jax.nn.sigmoid(_mm(h, w2[...]) + b2[...])
    gr_ref[...] = gr
    he = _bn_relu(_mm(gr, ew1[...]) + eb1[...], eg1[...], ebe1[...])
    env_ref[...] = _mm(he, ew2[...]) + eb2[...]


def _graph_heads(s, c, y, p):
    hdim = p['lfenv_W2'].shape[1]
    w1 = p['lfenv_W1']
    args = (s, c, y,
            w1[:hdim], w1[hdim:], p['lfenv_b1'].reshape(1, -1),
            p['lfenv_g1'].reshape(1, -1), p['lfenv_be1'].reshape(1, -1),
            p['lfenv_W2'], p['lfenv_b2'].reshape(1, -1),
            p['envp_W1'], p['envp_b1'].reshape(1, -1),
            p['envp_g1'].reshape(1, -1), p['envp_be1'].reshape(1, -1),
            p['envp_W2'], p['envp_b2'].reshape(1, -1))
    return pl.pallas_call(
        _heads_body,
        out_shape=[
            jax.ShapeDtypeStruct((NGRAPH, hdim), jnp.float32),
            jax.ShapeDtypeStruct((NGRAPH, p['envp_W2'].shape[1]),
                                 jnp.float32),
        ],
    )(*args)


def _lfinv_a_body(h_ref, b3_ref, y_ref, w1a, w1b, b1, h1_ref, ss_ref,
                  sq_ref):
    i = pl.program_id(0)
    bvec = b3_ref[0, 0, :]
    onehot = (bvec[:, None] == jax.lax.broadcasted_iota(
        jnp.int32, (1, NGRAPH), 1)).astype(jnp.float32)
    yb = _mm_exact(onehot, y_ref[...])
    h1 = _mm(h_ref[...], w1a[...]) + _mm(yb, w1b[...]) + b1[...]
    h1_ref[...] = h1

    @pl.when(i == 0)
    def _():
        ss_ref[...] = jnp.zeros_like(ss_ref)
        sq_ref[...] = jnp.zeros_like(sq_ref)

    ss_ref[...] += jnp.sum(h1, axis=0, keepdims=True)
    sq_ref[...] += jnp.sum(h1 * h1, axis=0, keepdims=True)


def _lfinv_a(h, col, batch3, y, p):
    n = h.shape[0]
    w1 = p['lfinv_W1']
    d = w1.shape[0] - y.shape[1]
    hh = w1.shape[1]
    nb = n // BR
    return pl.pallas_call(
        _lfinv_a_body,
        grid=(nb,),
        in_specs=[
            pl.BlockSpec((BR, d), lambda i: (i, col)),
            pl.BlockSpec((1, 1, BR), lambda i: (i, 0, 0)),
            pl.BlockSpec((NGRAPH, y.shape[1]), lambda i: (0, 0)),
            pl.BlockSpec((d, hh), lambda i: (0, 0)),
            pl.BlockSpec((y.shape[1], hh), lambda i: (0, 0)),
            pl.BlockSpec((1, hh), lambda i: (0, 0)),
        ],
        out_specs=[
            pl.BlockSpec((BR, hh), lambda i: (i, 0)),
            pl.BlockSpec((1, hh), lambda i: (0, 0)),
            pl.BlockSpec((1, hh), lambda i: (0, 0)),
        ],
        out_shape=[
            jax.ShapeDtypeStruct((n, hh), jnp.float32),
            jax.ShapeDtypeStruct((1, hh), jnp.float32),
            jax.ShapeDtypeStruct((1, hh), jnp.float32),
        ],
    )(h, batch3, y, w1[:d], w1[d:], p['lfinv_b1'].reshape(1, hh))


def _lfinv_b_body(h1_ref, ss_ref, sq_ref, g1, be1, w2, b2, o_ref, *,
                  n_rows):
    m = ss_ref[...] / n_rows
    v = sq_ref[...] / n_rows - m * m
    hn = jnp.maximum(g1[...] * (h1_ref[...] - m) / jnp.sqrt(v + 1e-5)
                     + be1[...], 0.0)
    o_ref[...] = jax.nn.sigmoid(_mm(hn, w2[...]) + b2[...])


def _lfinv_b(h1, ss, sq, p):
    n, hh = h1.shape
    w2 = p['lfinv_W2']
    dout = w2.shape[1]
    nb = n // BR
    return pl.pallas_call(
        functools.partial(_lfinv_b_body, n_rows=float(n)),
        grid=(nb,),
        in_specs=[
            pl.BlockSpec((BR, hh), lambda i: (i, 0)),
            pl.BlockSpec((1, hh), lambda i: (0, 0)),
            pl.BlockSpec((1, hh), lambda i: (0, 0)),
            pl.BlockSpec((1, hh), lambda i: (0, 0)),
            pl.BlockSpec((1, hh), lambda i: (0, 0)),
            pl.BlockSpec((hh, dout), lambda i: (0, 0)),
            pl.BlockSpec((1, dout), lambda i: (0, 0)),
        ],
        out_specs=pl.BlockSpec((BR, dout), lambda i: (i, 0)),
        out_shape=jax.ShapeDtypeStruct((n, dout), jnp.float32),
    )(h1, ss, sq, p['lfinv_g1'].reshape(1, hh), p['lfinv_be1'].reshape(1, hh),
      w2, p['lfinv_b2'].reshape(1, dout))


_EXTS = ('fg', 'eg', 'ig')


def _stacked(params, fmt):
    return jnp.stack([params[fmt % ext] for ext in _EXTS])


def _gin3(params, x, src, dst, n):
    h = x
    for l in range(L):
        agg = _edge_segment_sum(h, src, dst, n)
        eps3 = jnp.stack([params['%s_eps_%d' % (ext, l)] for ext in _EXTS])
        h = _gin3_layer(h, agg, eps3.astype(jnp.float32),
                        _stacked(params, '%%s_W1_%d' % l),
                        _stacked(params, '%%s_b1_%d' % l),
                        _stacked(params, '%%s_W2_%d' % l),
                        _stacked(params, '%%s_b2_%d' % l),
                        shared_in=(l == 0))
    return h


def kernel(x, edge_index, batch, params):
    n = x.shape[0]
    src = edge_index[0]
    dst = edge_index[1]
    batch3 = batch.astype(jnp.int32).reshape(n // BR, 1, BR)

    h3 = _gin3(params, x, src, dst, n)
    hh = h3.shape[1] // 3

    s3, c = _pool_sums(h3, batch3)
    y = _outmlp(s3[:, :hh], c, params)
    graph_repr, env_pred = _graph_heads(s3[:, hh:2 * hh], c, y, params)

    h1, ss, sq = _lfinv_a(h3, 2, batch3, y, params)
    node_score = _lfinv_b(h1, ss, sq, params)

    return (y, y, graph_repr, node_score, env_pred)
